# baseline (device time: 20536 ns/iter reference)
import jax
import jax.numpy as jnp
from jax import lax
from jax.experimental import pallas as pl
from jax.experimental.pallas import tpu as pltpu

N_DEV = 4


def kernel(t, W):
    m, k = t.shape
    _, n = W.shape

    def body(t_ref, w_ref, out_ref, sbuf, rbuf, send_sems, recv_sems):
        my_pos = lax.axis_index("i")
        partner0 = my_pos ^ 1
        partner1 = (N_DEV - 1) - my_pos

        sbuf[:, :] = jnp.dot(
            t_ref[:, :], w_ref[:, :], preferred_element_type=jnp.float32
        )

        barrier_sem = pltpu.get_barrier_semaphore()
        for nbr in [partner0, partner1]:
            pl.semaphore_signal(
                barrier_sem, inc=1,
                device_id=(nbr,), device_id_type=pl.DeviceIdType.MESH,
            )
        pl.semaphore_wait(barrier_sem, 2)

        rdma0 = pltpu.make_async_remote_copy(
            src_ref=sbuf,
            dst_ref=rbuf.at[0],
            send_sem=send_sems.at[0],
            recv_sem=recv_sems.at[0],
            device_id=(partner0,),
            device_id_type=pl.DeviceIdType.MESH,
        )
        rdma0.start()
        rdma0.wait()
        sbuf[:, :] = sbuf[:, :] + rbuf[0, :, :]

        rdma1 = pltpu.make_async_remote_copy(
            src_ref=sbuf,
            dst_ref=rbuf.at[1],
            send_sem=send_sems.at[1],
            recv_sem=recv_sems.at[1],
            device_id=(partner1,),
            device_id_type=pl.DeviceIdType.MESH,
        )
        rdma1.start()
        rdma1.wait()
        out_ref[:, :] = sbuf[:, :] + rbuf[1, :, :]

    return pl.pallas_call(
        body,
        out_shape=jax.ShapeDtypeStruct((m, n), jnp.float32),
        in_specs=[
            pl.BlockSpec(memory_space=pltpu.VMEM),
            pl.BlockSpec(memory_space=pltpu.VMEM),
        ],
        out_specs=pl.BlockSpec(memory_space=pltpu.VMEM),
        scratch_shapes=[
            pltpu.VMEM((m, n), jnp.float32),
            pltpu.VMEM((2, m, n), jnp.float32),
            pltpu.SemaphoreType.DMA((2,)),
            pltpu.SemaphoreType.DMA((2,)),
        ],
        compiler_params=pltpu.CompilerParams(collective_id=0),
    )(t, W)


# device time: 16315 ns/iter; 1.2587x vs baseline; 1.2587x over previous
import jax
import jax.numpy as jnp
from jax import lax
from jax.experimental import pallas as pl
from jax.experimental.pallas import tpu as pltpu

N_DEV = 4
C = 4


def kernel(t, W):
    m, k = t.shape
    _, n = W.shape
    r = m // C

    def body(t_ref, w_ref, out_ref, sbuf0, rbuf0, sbuf1, rbuf1,
             ss0, rs0, ss1, rs1):
        my_pos = lax.axis_index("i")
        partner0 = my_pos ^ 1
        partner1 = (N_DEV - 1) - my_pos

        barrier_sem = pltpu.get_barrier_semaphore()
        for nbr in [partner0, partner1]:
            pl.semaphore_signal(
                barrier_sem, inc=1,
                device_id=(nbr,), device_id_type=pl.DeviceIdType.MESH,
            )
        pl.semaphore_wait(barrier_sem, 2)

        d0 = []
        for c in range(C):
            sbuf0[c, :, :] = jnp.dot(
                t_ref[pl.ds(c * r, r), :], w_ref[:, :],
                preferred_element_type=jnp.float32,
            )
            rdma = pltpu.make_async_remote_copy(
                src_ref=sbuf0.at[c],
                dst_ref=rbuf0.at[c],
                send_sem=ss0.at[c],
                recv_sem=rs0.at[c],
                device_id=(partner0,),
                device_id_type=pl.DeviceIdType.MESH,
            )
            rdma.start()
            d0.append(rdma)

        d1 = []
        for c in range(C):
            d0[c].wait_recv()
            sbuf1[c, :, :] = sbuf0[c, :, :] + rbuf0[c, :, :]
            rdma = pltpu.make_async_remote_copy(
                src_ref=sbuf1.at[c],
                dst_ref=rbuf1.at[c],
                send_sem=ss1.at[c],
                recv_sem=rs1.at[c],
                device_id=(partner1,),
                device_id_type=pl.DeviceIdType.MESH,
            )
            rdma.start()
            d1.append(rdma)

        for c in range(C):
            d1[c].wait_recv()
            out_ref[pl.ds(c * r, r), :] = sbuf1[c, :, :] + rbuf1[c, :, :]

        for c in range(C):
            d0[c].wait_send()
            d1[c].wait_send()

    return pl.pallas_call(
        body,
        out_shape=jax.ShapeDtypeStruct((m, n), jnp.float32),
        in_specs=[
            pl.BlockSpec(memory_space=pltpu.VMEM),
            pl.BlockSpec(memory_space=pltpu.VMEM),
        ],
        out_specs=pl.BlockSpec(memory_space=pltpu.VMEM),
        scratch_shapes=[
            pltpu.VMEM((C, r, n), jnp.float32),
            pltpu.VMEM((C, r, n), jnp.float32),
            pltpu.VMEM((C, r, n), jnp.float32),
            pltpu.VMEM((C, r, n), jnp.float32),
            pltpu.SemaphoreType.DMA((C,)),
            pltpu.SemaphoreType.DMA((C,)),
            pltpu.SemaphoreType.DMA((C,)),
            pltpu.SemaphoreType.DMA((C,)),
        ],
        compiler_params=pltpu.CompilerParams(collective_id=0),
    )(t, W)


# device time: 13849 ns/iter; 1.4829x vs baseline; 1.1781x over previous
import jax
import jax.numpy as jnp
from jax import lax
from jax.experimental import pallas as pl
from jax.experimental.pallas import tpu as pltpu

N_DEV = 4
C = 2


def kernel(t, W):
    m, k = t.shape
    _, n = W.shape
    r = m // (2 * C)

    def body(t_ref, w_ref, out_ref, sb1, rb1, sb2, rb2, s1s, s1r, s2s, s2r):
        my_pos = lax.axis_index("i")
        partner0 = my_pos ^ 1
        partner1 = (N_DEV - 1) - my_pos
        routes = [(partner0, partner1), (partner1, partner0)]

        barrier_sem = pltpu.get_barrier_semaphore()
        for nbr in [partner0, partner1]:
            pl.semaphore_signal(
                barrier_sem, inc=1,
                device_id=(nbr,), device_id_type=pl.DeviceIdType.MESH,
            )
        pl.semaphore_wait(barrier_sem, 2)

        def row0(h, c):
            return (h * C + c) * r

        d1 = {}
        for c in range(C):
            for h in range(2):
                sb1[h, c, :, :] = jnp.dot(
                    t_ref[pl.ds(row0(h, c), r), :], w_ref[:, :],
                    preferred_element_type=jnp.float32,
                )
                rdma = pltpu.make_async_remote_copy(
                    src_ref=sb1.at[h, c],
                    dst_ref=rb1.at[h, c],
                    send_sem=s1s.at[h, c],
                    recv_sem=s1r.at[h, c],
                    device_id=(routes[h][0],),
                    device_id_type=pl.DeviceIdType.MESH,
                )
                rdma.start()
                d1[h, c] = rdma

        d2 = {}
        for c in range(C):
            for h in range(2):
                d1[h, c].wait_recv()
                sb2[h, c, :, :] = sb1[h, c, :, :] + rb1[h, c, :, :]
                rdma = pltpu.make_async_remote_copy(
                    src_ref=sb2.at[h, c],
                    dst_ref=rb2.at[h, c],
                    send_sem=s2s.at[h, c],
                    recv_sem=s2r.at[h, c],
                    device_id=(routes[h][1],),
                    device_id_type=pl.DeviceIdType.MESH,
                )
                rdma.start()
                d2[h, c] = rdma

        for c in range(C):
            for h in range(2):
                d2[h, c].wait_recv()
                out_ref[pl.ds(row0(h, c), r), :] = (
                    sb2[h, c, :, :] + rb2[h, c, :, :]
                )

        for c in range(C):
            for h in range(2):
                d1[h, c].wait_send()
                d2[h, c].wait_send()

    return pl.pallas_call(
        body,
        out_shape=jax.ShapeDtypeStruct((m, n), jnp.float32),
        in_specs=[
            pl.BlockSpec(memory_space=pltpu.VMEM),
            pl.BlockSpec(memory_space=pltpu.VMEM),
        ],
        out_specs=pl.BlockSpec(memory_space=pltpu.VMEM),
        scratch_shapes=[
            pltpu.VMEM((2, C, r, n), jnp.float32),
            pltpu.VMEM((2, C, r, n), jnp.float32),
            pltpu.VMEM((2, C, r, n), jnp.float32),
            pltpu.VMEM((2, C, r, n), jnp.float32),
            pltpu.SemaphoreType.DMA((2, C)),
            pltpu.SemaphoreType.DMA((2, C)),
            pltpu.SemaphoreType.DMA((2, C)),
            pltpu.SemaphoreType.DMA((2, C)),
        ],
        compiler_params=pltpu.CompilerParams(collective_id=0),
    )(t, W)


# device time: 11248 ns/iter; 1.8257x vs baseline; 1.2312x over previous
import jax
import jax.numpy as jnp
from jax import lax
from jax.experimental import pallas as pl
from jax.experimental.pallas import tpu as pltpu

N_DEV = 4
C = 2


def kernel(t, W):
    m, k = t.shape
    _, n = W.shape
    r = m // (2 * C)

    def body(t_ref, w_ref, out_ref, wb, sb1, rb1, sb2, rb2,
             s1s, s1r, s2s, s2r):
        my_pos = lax.axis_index("i")
        partner0 = my_pos ^ 1
        partner1 = (N_DEV - 1) - my_pos
        routes = [(partner0, partner1), (partner1, partner0)]

        barrier_sem = pltpu.get_barrier_semaphore()
        for nbr in [partner0, partner1]:
            pl.semaphore_signal(
                barrier_sem, inc=1,
                device_id=(nbr,), device_id_type=pl.DeviceIdType.MESH,
            )

        wb[:, :] = w_ref[:, :].astype(jnp.bfloat16)

        def row0(h, c):
            return (h * C + c) * r

        def dot_chunk(h, c):
            sb1[h, c, :, :] = jnp.dot(
                t_ref[pl.ds(row0(h, c), r), :].astype(jnp.bfloat16),
                wb[:, :],
                preferred_element_type=jnp.float32,
            ).astype(jnp.bfloat16)

        def send1(h, c):
            rdma = pltpu.make_async_remote_copy(
                src_ref=sb1.at[h, c],
                dst_ref=rb1.at[h, c],
                send_sem=s1s.at[h, c],
                recv_sem=s1r.at[h, c],
                device_id=(routes[h][0],),
                device_id_type=pl.DeviceIdType.MESH,
            )
            rdma.start()
            d1[h, c] = rdma

        d1 = {}
        dot_chunk(0, 0)
        dot_chunk(1, 0)
        pl.semaphore_wait(barrier_sem, 2)
        send1(0, 0)
        send1(1, 0)
        for c in range(1, C):
            for h in range(2):
                dot_chunk(h, c)
                send1(h, c)

        d2 = {}
        for c in range(C):
            for h in range(2):
                d1[h, c].wait_recv()
                sb2[h, c, :, :] = (
                    sb1[h, c, :, :].astype(jnp.float32)
                    + rb1[h, c, :, :].astype(jnp.float32)
                ).astype(jnp.bfloat16)
                rdma = pltpu.make_async_remote_copy(
                    src_ref=sb2.at[h, c],
                    dst_ref=rb2.at[h, c],
                    send_sem=s2s.at[h, c],
                    recv_sem=s2r.at[h, c],
                    device_id=(routes[h][1],),
                    device_id_type=pl.DeviceIdType.MESH,
                )
                rdma.start()
                d2[h, c] = rdma

        for c in range(C):
            for h in range(2):
                d2[h, c].wait_recv()
                out_ref[pl.ds(row0(h, c), r), :] = (
                    sb2[h, c, :, :].astype(jnp.float32)
                    + rb2[h, c, :, :].astype(jnp.float32)
                )

        for c in range(C):
            for h in range(2):
                d1[h, c].wait_send()
                d2[h, c].wait_send()

    return pl.pallas_call(
        body,
        out_shape=jax.ShapeDtypeStruct((m, n), jnp.float32),
        in_specs=[
            pl.BlockSpec(memory_space=pltpu.VMEM),
            pl.BlockSpec(memory_space=pltpu.VMEM),
        ],
        out_specs=pl.BlockSpec(memory_space=pltpu.VMEM),
        scratch_shapes=[
            pltpu.VMEM((k, n), jnp.bfloat16),
            pltpu.VMEM((2, C, r, n), jnp.bfloat16),
            pltpu.VMEM((2, C, r, n), jnp.bfloat16),
            pltpu.VMEM((2, C, r, n), jnp.bfloat16),
            pltpu.VMEM((2, C, r, n), jnp.bfloat16),
            pltpu.SemaphoreType.DMA((2, C)),
            pltpu.SemaphoreType.DMA((2, C)),
            pltpu.SemaphoreType.DMA((2, C)),
            pltpu.SemaphoreType.DMA((2, C)),
        ],
        compiler_params=pltpu.CompilerParams(collective_id=0),
    )(t, W)
